# Initial kernel scaffold; baseline (speedup 1.0000x reference)
import jax
import jax.numpy as jnp
from jax.experimental import pallas as pl


def kernel(feat, edge_index, W_self, W_neigh):
    raise NotImplementedError("write your pallas kernel here")


# speedup vs baseline: 3.1283x; 3.1283x over previous
import functools

import jax
import jax.numpy as jnp
from jax import lax
from jax.experimental import pallas as pl
from jax.experimental.pallas import tpu as pltpu
from jax.experimental.pallas import tpu_sc as plsc

N = 10000
E = 160000
D = 256
H = D // 2
NS = 16
RPT = 632
NPAD = NS * RPT
CH = 128
EPT = 10240
EPAD = EPT * NS
NCH = EPT // CH
BLK = 2000


def _sc_body(feat_lo, feat_hi, src_hbm, dst_hbm, zacc, zdeg, ones_hbm,
             out_sum, out_deg,
             acc, dacc, src_v, dst_v, rows_v, ones_v, sem):
    c = lax.axis_index("c")
    s = lax.axis_index("s")
    r0 = s * RPT

    pltpu.sync_copy(zacc.at[pl.ds(r0, RPT)], acc.at[pl.ds(r0, RPT)])
    pltpu.sync_copy(zdeg.at[pl.ds(r0, RPT)], dacc.at[pl.ds(r0, RPT)])
    pltpu.sync_copy(ones_hbm, ones_v)
    plsc.subcore_barrier()

    def chunk(k, carry):
        base = s * EPT + k * CH
        pltpu.sync_copy(src_hbm.at[pl.ds(base, CH)], src_v)
        pltpu.sync_copy(dst_hbm.at[pl.ds(base, CH)], dst_v)

        @pl.when(c == 0)
        def _():
            pltpu.async_copy(feat_lo.at[src_v], rows_v, sem).wait()

        @pl.when(c == 1)
        def _():
            pltpu.async_copy(feat_hi.at[src_v], rows_v, sem).wait()

        pltpu.sync_copy(rows_v, acc.at[dst_v], add=True)

        deg_here = jnp.where(c == 0, k < NCH // 2, k >= NCH // 2)

        @pl.when(deg_here)
        def _():
            pltpu.sync_copy(ones_v, dacc.at[dst_v], add=True)

        return carry

    lax.fori_loop(0, NCH, chunk, 0)
    plsc.subcore_barrier()

    pltpu.sync_copy(acc.at[pl.ds(r0, RPT)], out_sum.at[c, pl.ds(r0, RPT)])
    pltpu.sync_copy(dacc.at[pl.ds(r0, RPT)], out_deg.at[c, pl.ds(r0, RPT)])


_sc_fn = pl.kernel(
    _sc_body,
    out_type=[
        jax.ShapeDtypeStruct((2, NPAD, H), jnp.float32),
        jax.ShapeDtypeStruct((2, NPAD, 16), jnp.float32),
    ],
    mesh=plsc.VectorSubcoreMesh(core_axis_name="c", subcore_axis_name="s"),
    scratch_types=[
        pltpu.VMEM_SHARED((NPAD, H), jnp.float32),
        pltpu.VMEM_SHARED((NPAD, 16), jnp.float32),
        pltpu.VMEM((CH,), jnp.int32),
        pltpu.VMEM((CH,), jnp.int32),
        pltpu.VMEM((CH, H), jnp.float32),
        pltpu.VMEM((CH, 16), jnp.float32),
        pltpu.SemaphoreType.DMA,
    ],
    compiler_params=pltpu.CompilerParams(use_tc_tiling_on_sc=False),
)


def _tc_body(feat_ref, slo_ref, shi_ref, d0_ref, d1_ref,
             wst_ref, wnl_ref, wnh_ref, out_ref):
    deg = d0_ref[:, 0:1] + d1_ref[:, 0:1]
    r = 1.0 / jnp.maximum(deg, 1.0)
    acc = jnp.dot(feat_ref[...], wst_ref[...],
                  preferred_element_type=jnp.float32)
    acc = acc + jnp.dot(slo_ref[...] * r, wnl_ref[...],
                        preferred_element_type=jnp.float32)
    acc = acc + jnp.dot(shi_ref[...] * r, wnh_ref[...],
                        preferred_element_type=jnp.float32)
    out_ref[...] = acc


_tc_fn = pl.pallas_call(
    _tc_body,
    grid=(N // BLK,),
    in_specs=[
        pl.BlockSpec((BLK, D), lambda i: (i, 0)),
        pl.BlockSpec((BLK, H), lambda i: (i, 0)),
        pl.BlockSpec((BLK, H), lambda i: (i, 0)),
        pl.BlockSpec((BLK, 16), lambda i: (i, 0)),
        pl.BlockSpec((BLK, 16), lambda i: (i, 0)),
        pl.BlockSpec((D, D), lambda i: (0, 0)),
        pl.BlockSpec((H, D), lambda i: (0, 0)),
        pl.BlockSpec((H, D), lambda i: (0, 0)),
    ],
    out_specs=pl.BlockSpec((BLK, D), lambda i: (i, 0)),
    out_shape=jax.ShapeDtypeStruct((N, D), jnp.float32),
)


def kernel(feat, edge_index, W_self, W_neigh):
    src = edge_index[0].astype(jnp.int32)
    dst = edge_index[1].astype(jnp.int32)
    pad = EPAD - E
    src_p = jnp.concatenate([src, jnp.zeros((pad,), jnp.int32)])
    dst_p = jnp.concatenate([dst, jnp.full((pad,), N + 8, jnp.int32)])
    feat_lo = feat[:, :H]
    feat_hi = feat[:, H:]
    zacc = jnp.zeros((NPAD, H), jnp.float32)
    zdeg = jnp.zeros((NPAD, 16), jnp.float32)
    ones = jnp.ones((CH, 16), jnp.float32)

    sums, degs = _sc_fn(feat_lo, feat_hi, src_p, dst_p, zacc, zdeg, ones)

    return _tc_fn(feat, sums[0], sums[1], degs[0], degs[1],
                  W_self.T, W_neigh.T[:H], W_neigh.T[H:])


# speedup vs baseline: 4.1702x; 1.3331x over previous
import functools

import jax
import jax.numpy as jnp
from jax import lax
from jax.experimental import pallas as pl
from jax.experimental.pallas import tpu as pltpu
from jax.experimental.pallas import tpu_sc as plsc

N = 10000
E = 160000
D = 256
H = D // 2
NS = 16
RPT = 632
NPAD = NS * RPT
CH = 64
EPT = 10240
EPAD = EPT * NS
NCH = EPT // CH
BLK = 2000


def _sc_body(feat_lo, feat_hi, src_hbm, dst_hbm, zacc, zdeg, ones_hbm,
             out_sum, out_deg,
             acc, dacc, src_v, dst_v, rows_a, rows_b, ones_v, sem_a, sem_b):
    c = lax.axis_index("c")
    s = lax.axis_index("s")
    r0 = s * RPT

    pltpu.sync_copy(zacc.at[pl.ds(r0, RPT)], acc.at[pl.ds(r0, RPT)])
    pltpu.sync_copy(zdeg.at[pl.ds(r0, RPT)], dacc.at[pl.ds(r0, RPT)])
    pltpu.sync_copy(src_hbm.at[s], src_v)
    pltpu.sync_copy(dst_hbm.at[s], dst_v)
    pltpu.sync_copy(ones_hbm, ones_v)
    plsc.subcore_barrier()

    feat_c = [feat_lo, feat_hi]

    def gather(k, rows, sem):
        @pl.when(c == 0)
        def _():
            pltpu.async_copy(feat_c[0].at[src_v.at[k]], rows, sem)

        @pl.when(c == 1)
        def _():
            pltpu.async_copy(feat_c[1].at[src_v.at[k]], rows, sem)

    def wait(rows, sem):
        pltpu.make_async_copy(feat_c[0].at[src_v.at[0]], rows, sem).wait()

    def scatter(k, rows):
        pltpu.sync_copy(rows, acc.at[dst_v.at[k]], add=True)
        deg_here = jnp.where(c == 0, k < NCH // 2, k >= NCH // 2)

        @pl.when(deg_here)
        def _():
            pltpu.sync_copy(ones_v, dacc.at[dst_v.at[k]], add=True)

    gather(0, rows_a, sem_a)
    gather(1, rows_b, sem_b)

    def pair(i, carry):
        k0 = 2 * i
        wait(rows_a, sem_a)
        scatter(k0, rows_a)

        @pl.when(k0 + 2 < NCH)
        def _():
            gather(k0 + 2, rows_a, sem_a)

        wait(rows_b, sem_b)
        scatter(k0 + 1, rows_b)

        @pl.when(k0 + 3 < NCH)
        def _():
            gather(k0 + 3, rows_b, sem_b)

        return carry

    lax.fori_loop(0, NCH // 2, pair, 0)
    plsc.subcore_barrier()

    pltpu.sync_copy(acc.at[pl.ds(r0, RPT)], out_sum.at[c, pl.ds(r0, RPT)])
    pltpu.sync_copy(dacc.at[pl.ds(r0, RPT)], out_deg.at[c, pl.ds(r0, RPT)])


_sc_fn = pl.kernel(
    _sc_body,
    out_type=[
        jax.ShapeDtypeStruct((2, NPAD, H), jnp.float32),
        jax.ShapeDtypeStruct((2, NPAD, 16), jnp.float32),
    ],
    mesh=plsc.VectorSubcoreMesh(core_axis_name="c", subcore_axis_name="s"),
    scratch_types=[
        pltpu.VMEM_SHARED((NPAD, H), jnp.float32),
        pltpu.VMEM_SHARED((NPAD, 16), jnp.float32),
        pltpu.VMEM((NCH, CH), jnp.int32),
        pltpu.VMEM((NCH, CH), jnp.int32),
        pltpu.VMEM((CH, H), jnp.float32),
        pltpu.VMEM((CH, H), jnp.float32),
        pltpu.VMEM((CH, 16), jnp.float32),
        pltpu.SemaphoreType.DMA,
        pltpu.SemaphoreType.DMA,
    ],
    compiler_params=pltpu.CompilerParams(use_tc_tiling_on_sc=False),
)


def _tc_body(feat_ref, slo_ref, shi_ref, d0_ref, d1_ref,
             wst_ref, wnl_ref, wnh_ref, out_ref):
    deg = d0_ref[:, 0:1] + d1_ref[:, 0:1]
    r = 1.0 / jnp.maximum(deg, 1.0)
    acc = jnp.dot(feat_ref[...], wst_ref[...],
                  preferred_element_type=jnp.float32)
    acc = acc + jnp.dot(slo_ref[...] * r, wnl_ref[...],
                        preferred_element_type=jnp.float32)
    acc = acc + jnp.dot(shi_ref[...] * r, wnh_ref[...],
                        preferred_element_type=jnp.float32)
    out_ref[...] = acc


_tc_fn = pl.pallas_call(
    _tc_body,
    grid=(N // BLK,),
    in_specs=[
        pl.BlockSpec((BLK, D), lambda i: (i, 0)),
        pl.BlockSpec((BLK, H), lambda i: (i, 0)),
        pl.BlockSpec((BLK, H), lambda i: (i, 0)),
        pl.BlockSpec((BLK, 16), lambda i: (i, 0)),
        pl.BlockSpec((BLK, 16), lambda i: (i, 0)),
        pl.BlockSpec((D, D), lambda i: (0, 0)),
        pl.BlockSpec((H, D), lambda i: (0, 0)),
        pl.BlockSpec((H, D), lambda i: (0, 0)),
    ],
    out_specs=pl.BlockSpec((BLK, D), lambda i: (i, 0)),
    out_shape=jax.ShapeDtypeStruct((N, D), jnp.float32),
)


def kernel(feat, edge_index, W_self, W_neigh):
    src = edge_index[0].astype(jnp.int32)
    dst = edge_index[1].astype(jnp.int32)
    pad = EPAD - E
    src_p = jnp.concatenate([src, jnp.zeros((pad,), jnp.int32)]).reshape(NS, NCH, CH)
    dst_p = jnp.concatenate([dst, jnp.full((pad,), N + 8, jnp.int32)]).reshape(NS, NCH, CH)
    feat_lo = feat[:, :H]
    feat_hi = feat[:, H:]
    zacc = jnp.zeros((NPAD, H), jnp.float32)
    zdeg = jnp.zeros((NPAD, 16), jnp.float32)
    ones = jnp.ones((CH, 16), jnp.float32)

    sums, degs = _sc_fn(feat_lo, feat_hi, src_p, dst_p, zacc, zdeg, ones)

    return _tc_fn(feat, sums[0], sums[1], degs[0], degs[1],
                  W_self.T, W_neigh.T[:H], W_neigh.T[H:])
